# Initial kernel scaffold; baseline (speedup 1.0000x reference)
import jax
import jax.numpy as jnp
from jax.experimental import pallas as pl


def kernel(x_words, x_sent, w2w_index, w2s_index, s2s_index, s2s_type, s2s_sim, params):
    raise NotImplementedError("write your pallas kernel here")


# speedup vs baseline: 21.4491x; 21.4491x over previous
import functools

import jax
import jax.numpy as jnp
from jax import lax
from jax.experimental import pallas as pl
from jax.experimental.pallas import tpu as pltpu
from jax.experimental.pallas import tpu_sc as plsc

_NW = 50000
_NS = 10000
_D = 128
_H = 8
_OUT = 4

_f32 = jnp.float32
_i32 = jnp.int32


def _mm2_body(y, wl, bl, wr, br, xl, xr):
    yv = y[...]
    xl[...] = jnp.dot(yv, wl[...], preferred_element_type=_f32) + bl[...]
    xr[...] = jnp.dot(yv, wr[...], preferred_element_type=_f32) + br[...]


def _mm2(y, p):
    n = y.shape[0]
    bn = 1000
    return pl.pallas_call(
        _mm2_body,
        grid=(n // bn,),
        in_specs=[
            pl.BlockSpec((bn, _D), lambda i: (i, 0)),
            pl.BlockSpec((_D, _D), lambda i: (0, 0)),
            pl.BlockSpec((1, _D), lambda i: (0, 0)),
            pl.BlockSpec((_D, _D), lambda i: (0, 0)),
            pl.BlockSpec((1, _D), lambda i: (0, 0)),
        ],
        out_specs=[pl.BlockSpec((bn, _D), lambda i: (i, 0))] * 2,
        out_shape=[jax.ShapeDtypeStruct((n, _D), _f32)] * 2,
    )(y, p['Wl'], p['bl'][None], p['Wr'], p['br'][None])


def _xe_body(ea, we, xe, cs):
    x = jnp.dot(ea[...], we[...], preferred_element_type=_f32)
    xe[...] = x

    @pl.when(pl.program_id(0) == 0)
    def _():
        cs[...] = jnp.zeros_like(cs)

    cs[...] += jnp.sum(x, axis=0, keepdims=True)


def _xe(ea_pad, we):
    ep = ea_pad.shape[0]
    be = 512
    return pl.pallas_call(
        _xe_body,
        grid=(ep // be,),
        in_specs=[
            pl.BlockSpec((be, 17), lambda i: (i, 0)),
            pl.BlockSpec((17, _D), lambda i: (0, 0)),
        ],
        out_specs=[
            pl.BlockSpec((be, _D), lambda i: (i, 0)),
            pl.BlockSpec((1, _D), lambda i: (0, 0)),
        ],
        out_shape=[
            jax.ShapeDtypeStruct((ep, _D), _f32),
            jax.ShapeDtypeStruct((1, _D), _f32),
        ],
    )(ea_pad, we)


def _combine_body(p0, p1, d0, d1, xl, xr, fill, attb, b8, bias, g, b, out):
    xlv = xl[...]
    m = xlv + xr[...] + fill[...]
    ma = jnp.maximum(m, 0.2 * m)
    exs = jnp.exp(jnp.dot(ma, attb[...], preferred_element_type=_f32))
    den = d0[...] + d1[...] + exs
    acc = p0[...] + p1[...] + xlv * jnp.dot(exs, b8[...], preferred_element_type=_f32)
    o = acc / (jnp.dot(den, b8[...], preferred_element_type=_f32) + 1e-16)
    o = o + bias[...]
    mu = jnp.mean(o, axis=1, keepdims=True)
    var = jnp.mean(jnp.square(o - mu), axis=1, keepdims=True)
    o = (o - mu) * lax.rsqrt(var + 1e-5) * g[...] + b[...]
    out[...] = jnp.maximum(o, 0.0)


def _combine(p0, p1, d0, d1, xl, xr, fill, attb, b8, p, lnp):
    n = xl.shape[0]
    bn = 1000
    return pl.pallas_call(
        _combine_body,
        grid=(n // bn,),
        in_specs=[
            pl.BlockSpec((bn, _D), lambda i: (i, 0)),
            pl.BlockSpec((bn, _D), lambda i: (i, 0)),
            pl.BlockSpec((bn, 8), lambda i: (i, 0)),
            pl.BlockSpec((bn, 8), lambda i: (i, 0)),
            pl.BlockSpec((bn, _D), lambda i: (i, 0)),
            pl.BlockSpec((bn, _D), lambda i: (i, 0)),
            pl.BlockSpec((1, _D), lambda i: (0, 0)),
            pl.BlockSpec((_D, _H), lambda i: (0, 0)),
            pl.BlockSpec((_H, _D), lambda i: (0, 0)),
            pl.BlockSpec((1, _D), lambda i: (0, 0)),
            pl.BlockSpec((1, _D), lambda i: (0, 0)),
            pl.BlockSpec((1, _D), lambda i: (0, 0)),
        ],
        out_specs=pl.BlockSpec((bn, _D), lambda i: (i, 0)),
        out_shape=jax.ShapeDtypeStruct((n, _D), _f32),
    )(p0, p1, d0, d1, xl, xr, fill, attb, b8, p['bias'][None],
      lnp['g'][None], lnp['b'][None])


def _cls_body(y, w1, b1, g, b, w2, b2, out):
    h = jnp.dot(y[...], w1[...], preferred_element_type=_f32) + b1[...]
    mu = jnp.mean(h, axis=1, keepdims=True)
    var = jnp.mean(jnp.square(h - mu), axis=1, keepdims=True)
    h = (h - mu) * lax.rsqrt(var + 1e-5) * g[...] + b[...]
    h = jnp.maximum(h, 0.0)
    out[...] = jnp.dot(h, w2[...], preferred_element_type=_f32) + b2[...]


def _cls(y, params):
    n = y.shape[0]
    bn = 1000
    dq = _D // 4
    return pl.pallas_call(
        _cls_body,
        grid=(n // bn,),
        in_specs=[
            pl.BlockSpec((bn, _D), lambda i: (i, 0)),
            pl.BlockSpec((_D, dq), lambda i: (0, 0)),
            pl.BlockSpec((1, dq), lambda i: (0, 0)),
            pl.BlockSpec((1, dq), lambda i: (0, 0)),
            pl.BlockSpec((1, dq), lambda i: (0, 0)),
            pl.BlockSpec((dq, _OUT), lambda i: (0, 0)),
            pl.BlockSpec((1, _OUT), lambda i: (0, 0)),
        ],
        out_specs=pl.BlockSpec((bn, _OUT), lambda i: (i, 0)),
        out_shape=jax.ShapeDtypeStruct((n, _OUT), _f32),
    )(y, params['cls_W1'], params['cls_b1'][None], params['cls_ln']['g'][None],
      params['cls_ln']['b'][None], params['cls_W2'], params['cls_b2'][None])


def _edge_call(src, dst, xl, xr, xe, att2, ep, n, nr, r_cnt):
    has_xe = xe is not None
    ew = ep // 32
    ng = ew // 16
    maxc = ((ew + 63) // 64) * 64 + 128
    nrp = nr + 512
    rpt = nrp // 16
    zb = 8
    nd = nr // 16
    npad = nr * r_cnt

    mesh = plsc.VectorSubcoreMesh(core_axis_name="c", subcore_axis_name="s")

    scratch = [
        pltpu.VMEM((ew,), _i32),
        pltpu.VMEM((ew,), _i32),
        pltpu.VMEM((maxc,), _i32),
        pltpu.VMEM((maxc,), _i32),
        pltpu.VMEM((64,), _i32),
        pltpu.VMEM((1, 64), _i32),
        pltpu.VMEM((1, 64), _i32),
        pltpu.VMEM((64, _D), _f32),
        pltpu.VMEM((64, _D), _f32),
        pltpu.VMEM((64, _D), _f32),
        pltpu.VMEM((64, _D), _f32),
        pltpu.VMEM((_H, 16), _f32),
        pltpu.VMEM((zb, _D), _f32),
        pltpu.VMEM_SHARED((nrp, _D), _f32),
        pltpu.VMEM_SHARED((nr // 16 + 128, _D), _f32),
        pltpu.SemaphoreType.DMA,
        pltpu.SemaphoreType.DMA,
        pltpu.SemaphoreType.DMA,
    ]
    if has_xe:
        scratch = scratch[:7] + [pltpu.VMEM((maxc,), _i32),
                                 pltpu.VMEM((64, _D), _f32)] + scratch[7:]

    def body(*refs):
        if has_xe:
            (src_h, dst_h, xl_h, xr_h, xe_h, att_h, outp_h, denp_h,
             srcv, dstv, csrc, cdstl, cidx, cdst2, cdst3, ceid, xeb,
             xlb, xrb, sbuf, sbuf2, attv, zbuf, acc, acc2,
             sem1, sem2, sem3) = refs
        else:
            (src_h, dst_h, xl_h, xr_h, att_h, outp_h, denp_h,
             srcv, dstv, csrc, cdstl, cidx, cdst2, cdst3,
             xlb, xrb, sbuf, sbuf2, attv, zbuf, acc, acc2,
             sem1, sem2, sem3) = refs

        cid = lax.axis_index("c")
        sid = lax.axis_index("s")
        wid = sid * 2 + cid
        base = wid * ew
        iota16 = lax.iota(_i32, 16)
        zeros16 = jnp.zeros((16,), _f32)
        shuf_idx = [jnp.maximum(iota16 - k, 0)[:, None] for k in (1, 2, 4, 8)]
        shuf_keep = [iota16 >= k for k in (1, 2, 4, 8)]
        trash = maxc - 16 + iota16
        gd = lax.GatherDimensionNumbers(
            offset_dims=(), collapsed_slice_dims=(0,), start_index_map=(0,))

        def _lane_shift(v, idx):
            return lax.gather(v, idx, gd, (1,),
                              mode=lax.GatherScatterMode.PROMISE_IN_BOUNDS)

        pltpu.sync_copy(src_h.at[pl.ds(base, ew)], srcv)
        pltpu.sync_copy(dst_h.at[pl.ds(base, ew)], dstv)
        pltpu.sync_copy(att_h, attv)

        def _zrow(i, c):
            for j in range(8):
                zbuf[i, pl.ds(16 * j, 16)] = zeros16
            return c
        lax.fori_loop(0, zb, _zrow, 0)

        def _spad(e, c):
            for j in range(8):
                sbuf2[e, pl.ds(16 * j, 16)] = zeros16
            return c
        lax.fori_loop(0, 64, _spad, 0)

        def _range_body(r, carry):
            lo = r * nr

            def _zc(i, c):
                pltpu.sync_copy(zbuf, acc.at[pl.ds(sid * rpt + i * zb, zb)])
                return c
            lax.fori_loop(0, rpt // zb, _zc, 0)

            def _zc2(i, c):
                pltpu.sync_copy(
                    zbuf, acc2.at[pl.ds(sid * ((nd + 128) // 16) + i * zb, zb)])
                return c
            lax.fori_loop(0, (nd + 128) // 16 // zb, _zc2, 0)
            plsc.subcore_barrier()

            def _cg(g, cnt):
                d = dstv[pl.ds(g * 16, 16)]
                s = srcv[pl.ds(g * 16, 16)]
                dl = d - lo
                msk = (dl >= 0) & (dl < nr)
                pf = jnp.where(msk, 1, 0).astype(_i32)
                for k in range(4):
                    sh = _lane_shift(pf, shuf_idx[k])
                    pf = pf + jnp.where(shuf_keep[k], sh, 0)
                pos = jnp.where(msk, cnt + pf - 1, trash)
                plsc.store_scatter(csrc, [pos], s)
                plsc.store_scatter(cdstl, [pos], dl)
                if has_xe:
                    plsc.store_scatter(ceid, [pos], base + g * 16 + iota16)
                return cnt + pf[15]
            cnt = lax.fori_loop(0, ng, _cg, jnp.array(0, _i32))

            padv = jnp.full((16,), nr, _i32)
            zi = jnp.zeros((16,), _i32)
            for t in range(4):
                csrc[pl.ds(cnt + t * 16, 16)] = zi
                cdstl[pl.ds(cnt + t * 16, 16)] = padv
                if has_xe:
                    ceid[pl.ds(cnt + t * 16, 16)] = zi
            nchunks = (cnt + 63) // 64

            def _ck(k, c):
                for j in range(4):
                    dlj = cdstl[pl.ds(k * 64 + 16 * j, 16)]
                    cdst2[0, pl.ds(16 * j, 16)] = dlj
                    cdst3[0, pl.ds(16 * j, 16)] = dlj >> 4
                    cidx[pl.ds(16 * j, 16)] = dlj + lo
                cp1 = pltpu.async_copy(xl_h.at[csrc.at[pl.ds(k * 64, 64)]],
                                       xlb, sem1)
                cp2 = pltpu.async_copy(xr_h.at[cidx], xrb, sem2)
                if has_xe:
                    cp3 = pltpu.async_copy(xe_h.at[ceid.at[pl.ds(k * 64, 64)]],
                                           xeb, sem3)
                cp1.wait()
                cp2.wait()
                if has_xe:
                    cp3.wait()

                for h in range(8):
                    h16 = jnp.full((16,), h, _i32)
                    asp = [plsc.load_gather(attv, [h16, jnp.full((16,), cc, _i32)])
                           for cc in range(16)]

                    def _sg(s2, cc):
                        eid16 = s2 * 16 + iota16
                        al = jnp.zeros((16,), _f32)
                        for ccc in range(16):
                            col = jnp.full((16,), h * 16 + ccc, _i32)
                            mv = (plsc.load_gather(xlb, [eid16, col]) +
                                  plsc.load_gather(xrb, [eid16, col]))
                            if has_xe:
                                mv = mv + plsc.load_gather(xeb, [eid16, col])
                            mav = jnp.maximum(mv, 0.2 * mv)
                            al = al + mav * asp[ccc]
                        ex = jnp.exp(al)
                        dlj = cdstl[pl.ds(k * 64 + s2 * 16, 16)]
                        colv = ((dlj & 15) << 3) + h
                        plsc.store_scatter(sbuf2, [eid16, colv], ex)
                        return cc
                    lax.fori_loop(0, 4, _sg, 0)

                def _we(e, cc):
                    e16 = jnp.full((16,), 0, _i32) + e
                    dle = plsc.load_gather(cdst2, [jnp.zeros((16,), _i32), e16])
                    cbase = (dle & 15) << 3
                    for h in range(8):
                        exsp = plsc.load_gather(sbuf2, [e16, cbase + h])
                        sbuf[e, pl.ds(h * 16, 16)] = xlb[e, pl.ds(h * 16, 16)] * exsp
                    return cc
                lax.fori_loop(0, 64, _we, 0)

                pltpu.sync_copy(sbuf, acc.at[cdst2.at[0]], add=True)
                pltpu.sync_copy(sbuf2, acc2.at[cdst3.at[0]], add=True)

                def _zs(s2, cc):
                    eid16 = s2 * 16 + iota16
                    dlj = cdstl[pl.ds(k * 64 + s2 * 16, 16)]
                    cbase = (dlj & 15) << 3
                    for h in range(8):
                        plsc.store_scatter(sbuf2, [eid16, cbase + h], zeros16)
                    return cc
                lax.fori_loop(0, 4, _zs, 0)
                return c
            lax.fori_loop(0, nchunks, _ck, 0)

            plsc.subcore_barrier()

            row0 = sid * (nr // 16)
            pltpu.sync_copy(acc.at[pl.ds(row0, nr // 16)],
                            outp_h.at[cid, pl.ds(lo + row0, nr // 16)])
            row0d = sid * (nd // 16)
            pltpu.sync_copy(acc2.at[pl.ds(row0d, nd // 16)],
                            denp_h.at[cid, pl.ds(r * nd + row0d, nd // 16)])
            plsc.subcore_barrier()
            return carry
        lax.fori_loop(0, r_cnt, _range_body, 0)

    out_type = [
        jax.ShapeDtypeStruct((2, npad, 128), _f32),
        jax.ShapeDtypeStruct((2, npad // 16, 128), _f32),
    ]
    args = [src, dst, xl, xr]
    if has_xe:
        args.append(xe)
    args.append(att2)
    return pl.kernel(
        body, out_type=out_type, mesh=mesh, scratch_types=scratch,
        compiler_params=pltpu.CompilerParams(needs_layout_passes=False),
    )(*args)


def _pad_edges(ei, n, ep):
    e = ei.shape[1]
    src = jnp.concatenate([ei[0], jnp.zeros((ep - e,), _i32)])
    dst = jnp.concatenate([ei[1], jnp.full((ep - e,), n, _i32)])
    return src, dst


def _gat_layer(p, y, src, dst, ep, n, nr, r_cnt, b8, xe=None, fill=None):
    xl, xr = _mm2(y, p)
    att2 = p['att'].reshape(_H, 16)
    attb = jnp.zeros((_D, _H), _f32).at[
        jnp.arange(_D), jnp.arange(_D) // 16].set(p['att'].reshape(_D))
    outp, denp = _edge_call(src, dst, xl, xr, xe, att2, ep, n, nr, r_cnt)
    denr = denp.reshape(2, nr * r_cnt, 8)
    if fill is None:
        fill = jnp.zeros((1, _D), _f32)
    return _combine(outp[0, :n], outp[1, :n], denr[0, :n], denr[1, :n],
                    xl, xr, fill, attb, b8, p, p['_ln'])


def kernel(x_words, x_sent, w2w_index, w2s_index, s2s_index, s2s_type,
           s2s_sim, params):
    b8 = (jnp.arange(_D)[None, :] // 16 ==
          jnp.arange(_H)[:, None]).astype(_f32)

    ew_p = ((w2w_index.shape[1] + 511) // 512) * 512
    ews_p = ((w2s_index.shape[1] + 511) // 512) * 512
    es_p = ((s2s_index.shape[1] + 511) // 512) * 512

    w2w_s, w2w_d = _pad_edges(w2w_index, _NW, ew_p)
    w2s_s, w2s_d = _pad_edges(w2s_index, _NW + _NS, ews_p)
    s2s_s, s2s_d = _pad_edges(s2s_index, _NS, es_p)
    sim_s, sim_d = _pad_edges(s2s_sim, _NS, es_p)

    es = s2s_index.shape[1]
    ea_pad = jnp.concatenate(
        [s2s_type, jnp.zeros((es_p - es, 17), _f32)], axis=0)

    def lp(name, ln):
        q = dict(params[name])
        q['_ln'] = params[ln]
        return q

    y = _gat_layer(lp('w2w_1', 'ln1'), x_words, w2w_s, w2w_d,
                   ew_p, _NW, 4096, 13, b8)
    y = _gat_layer(lp('w2w_1', 'ln2'), y, w2w_s, w2w_d,
                   ew_p, _NW, 4096, 13, b8)
    yc = jnp.concatenate([y, x_sent], axis=0)
    yc = _gat_layer(lp('word_to_sent', 'ln3'), yc, w2s_s, w2s_d,
                    ews_p, _NW + _NS, 4096, 15, b8)
    ys = yc[_NW:]

    xe1, cs1 = _xe(ea_pad, params['s2s_1']['We'])
    ys = _gat_layer(lp('s2s_1', 'ln4'), ys, s2s_s, s2s_d,
                    es_p, _NS, 4096, 3, b8, xe=xe1, fill=cs1 / es)
    xe2, cs2 = _xe(ea_pad, params['s2s_2']['We'])
    ys = _gat_layer(lp('s2s_2', 'ln5'), ys, s2s_s, s2s_d,
                    es_p, _NS, 4096, 3, b8, xe=xe2, fill=cs2 / es)

    ys = _gat_layer(lp('red_1', 'ln6'), ys, sim_s, sim_d,
                    es_p, _NS, 4096, 3, b8)
    ys = _gat_layer(lp('red_2', 'ln7'), ys, sim_s, sim_d,
                    es_p, _NS, 4096, 3, b8)

    return _cls(ys, params)


# speedup vs baseline: 42.2637x; 1.9704x over previous
import functools

import jax
import jax.numpy as jnp
from jax import lax
from jax.experimental import pallas as pl
from jax.experimental.pallas import tpu as pltpu
from jax.experimental.pallas import tpu_sc as plsc

_NW = 50000
_NS = 10000
_D = 128
_H = 8
_OUT = 4

_f32 = jnp.float32
_i32 = jnp.int32


def _mm2_body(y, wl, bl, wr, br, xl, xr):
    yv = y[...]
    xl[...] = jnp.dot(yv, wl[...], preferred_element_type=_f32) + bl[...]
    xr[...] = jnp.dot(yv, wr[...], preferred_element_type=_f32) + br[...]


def _mm2(y, p):
    n = y.shape[0]
    bn = 1000
    return pl.pallas_call(
        _mm2_body,
        grid=(n // bn,),
        in_specs=[
            pl.BlockSpec((bn, _D), lambda i: (i, 0)),
            pl.BlockSpec((_D, _D), lambda i: (0, 0)),
            pl.BlockSpec((1, _D), lambda i: (0, 0)),
            pl.BlockSpec((_D, _D), lambda i: (0, 0)),
            pl.BlockSpec((1, _D), lambda i: (0, 0)),
        ],
        out_specs=[pl.BlockSpec((bn, _D), lambda i: (i, 0))] * 2,
        out_shape=[jax.ShapeDtypeStruct((n, _D), _f32)] * 2,
    )(y, p['Wl'], p['bl'][None], p['Wr'], p['br'][None])


def _xe_body(ea, we, xe, cs):
    x = jnp.dot(ea[...], we[...], preferred_element_type=_f32)
    xe[...] = x

    @pl.when(pl.program_id(0) == 0)
    def _():
        cs[...] = jnp.zeros_like(cs)

    cs[...] += jnp.sum(x, axis=0, keepdims=True)


def _xe(ea_pad, we):
    ep = ea_pad.shape[0]
    be = 512
    return pl.pallas_call(
        _xe_body,
        grid=(ep // be,),
        in_specs=[
            pl.BlockSpec((be, 17), lambda i: (i, 0)),
            pl.BlockSpec((17, _D), lambda i: (0, 0)),
        ],
        out_specs=[
            pl.BlockSpec((be, _D), lambda i: (i, 0)),
            pl.BlockSpec((1, _D), lambda i: (0, 0)),
        ],
        out_shape=[
            jax.ShapeDtypeStruct((ep, _D), _f32),
            jax.ShapeDtypeStruct((1, _D), _f32),
        ],
    )(ea_pad, we)


def _combine_body(p0, p1, d0, d1, xl, xr, fill, attb, b8, bias, g, b, out):
    xlv = xl[...]
    m = xlv + xr[...] + fill[...]
    ma = jnp.maximum(m, 0.2 * m)
    exs = jnp.exp(jnp.dot(ma, attb[...], preferred_element_type=_f32))
    den = d0[...][:, :8] + d1[...][:, :8] + exs
    acc = p0[...] + p1[...] + xlv * jnp.dot(exs, b8[...], preferred_element_type=_f32)
    o = acc / (jnp.dot(den, b8[...], preferred_element_type=_f32) + 1e-16)
    o = o + bias[...]
    mu = jnp.mean(o, axis=1, keepdims=True)
    var = jnp.mean(jnp.square(o - mu), axis=1, keepdims=True)
    o = (o - mu) * lax.rsqrt(var + 1e-5) * g[...] + b[...]
    out[...] = jnp.maximum(o, 0.0)


def _combine(p0, p1, d0, d1, xl, xr, fill, attb, b8, p, lnp):
    n = xl.shape[0]
    bn = 1000
    return pl.pallas_call(
        _combine_body,
        grid=(n // bn,),
        in_specs=[
            pl.BlockSpec((bn, _D), lambda i: (i, 0)),
            pl.BlockSpec((bn, _D), lambda i: (i, 0)),
            pl.BlockSpec((bn, 16), lambda i: (i, 0)),
            pl.BlockSpec((bn, 16), lambda i: (i, 0)),
            pl.BlockSpec((bn, _D), lambda i: (i, 0)),
            pl.BlockSpec((bn, _D), lambda i: (i, 0)),
            pl.BlockSpec((1, _D), lambda i: (0, 0)),
            pl.BlockSpec((_D, _H), lambda i: (0, 0)),
            pl.BlockSpec((_H, _D), lambda i: (0, 0)),
            pl.BlockSpec((1, _D), lambda i: (0, 0)),
            pl.BlockSpec((1, _D), lambda i: (0, 0)),
            pl.BlockSpec((1, _D), lambda i: (0, 0)),
        ],
        out_specs=pl.BlockSpec((bn, _D), lambda i: (i, 0)),
        out_shape=jax.ShapeDtypeStruct((n, _D), _f32),
    )(p0, p1, d0, d1, xl, xr, fill, attb, b8, p['bias'][None],
      lnp['g'][None], lnp['b'][None])


def _cls_body(y, w1, b1, g, b, w2, b2, out):
    h = jnp.dot(y[...], w1[...], preferred_element_type=_f32) + b1[...]
    mu = jnp.mean(h, axis=1, keepdims=True)
    var = jnp.mean(jnp.square(h - mu), axis=1, keepdims=True)
    h = (h - mu) * lax.rsqrt(var + 1e-5) * g[...] + b[...]
    h = jnp.maximum(h, 0.0)
    out[...] = jnp.dot(h, w2[...], preferred_element_type=_f32) + b2[...]


def _cls(y, params):
    n = y.shape[0]
    bn = 1000
    dq = _D // 4
    return pl.pallas_call(
        _cls_body,
        grid=(n // bn,),
        in_specs=[
            pl.BlockSpec((bn, _D), lambda i: (i, 0)),
            pl.BlockSpec((_D, dq), lambda i: (0, 0)),
            pl.BlockSpec((1, dq), lambda i: (0, 0)),
            pl.BlockSpec((1, dq), lambda i: (0, 0)),
            pl.BlockSpec((1, dq), lambda i: (0, 0)),
            pl.BlockSpec((dq, _OUT), lambda i: (0, 0)),
            pl.BlockSpec((1, _OUT), lambda i: (0, 0)),
        ],
        out_specs=pl.BlockSpec((bn, _OUT), lambda i: (i, 0)),
        out_shape=jax.ShapeDtypeStruct((n, _OUT), _f32),
    )(y, params['cls_W1'], params['cls_b1'][None], params['cls_ln']['g'][None],
      params['cls_ln']['b'][None], params['cls_W2'], params['cls_b2'][None])


def _edge_call(src, dst, xl, xr, xe, att2, ep, n, nr, r_cnt):
    has_xe = xe is not None
    ew = ep // 32
    ng = ew // 16
    maxc = ((ew + 63) // 64) * 64 + 128
    nrp = nr + 512
    rpt = nrp // 16
    zb = 8
    nd = nr // 8
    npad = nr * r_cnt

    mesh = plsc.VectorSubcoreMesh(core_axis_name="c", subcore_axis_name="s")

    scratch = [
        pltpu.VMEM((ew,), _i32),
        pltpu.VMEM((ew,), _i32),
        pltpu.VMEM((maxc,), _i32),
        pltpu.VMEM((maxc,), _i32),
        pltpu.VMEM((64,), _i32),
        pltpu.VMEM((1, 64), _i32),
        pltpu.VMEM((1, 64), _i32),
        pltpu.VMEM((64, _D), _f32),
        pltpu.VMEM((64, _D), _f32),
        pltpu.VMEM((64, _D), _f32),
        pltpu.VMEM((64, _D), _f32),
        pltpu.VMEM((_H, 16), _f32),
        pltpu.VMEM((zb, _D), _f32),
        pltpu.VMEM_SHARED((nrp, _D), _f32),
        pltpu.VMEM_SHARED((nr // 8 + 128, _D), _f32),
        pltpu.SemaphoreType.DMA,
        pltpu.SemaphoreType.DMA,
        pltpu.SemaphoreType.DMA,
    ]
    if has_xe:
        scratch = scratch[:7] + [pltpu.VMEM((maxc,), _i32),
                                 pltpu.VMEM((64, _D), _f32)] + scratch[7:]

    def body(*refs):
        if has_xe:
            (src_h, dst_h, xl_h, xr_h, xe_h, att_h, outp_h, denp_h,
             srcv, dstv, csrc, cdstl, cidx, cdst2, cdst3, ceid, xeb,
             xlb, xrb, sbuf, sbuf2, attv, zbuf, acc, acc2,
             sem1, sem2, sem3) = refs
        else:
            (src_h, dst_h, xl_h, xr_h, att_h, outp_h, denp_h,
             srcv, dstv, csrc, cdstl, cidx, cdst2, cdst3,
             xlb, xrb, sbuf, sbuf2, attv, zbuf, acc, acc2,
             sem1, sem2, sem3) = refs

        cid = lax.axis_index("c")
        sid = lax.axis_index("s")
        wid = sid * 2 + cid
        base = wid * ew
        iota16 = lax.iota(_i32, 16)
        zeros16 = jnp.zeros((16,), _f32)
        shuf_idx = [jnp.maximum(iota16 - k, 0)[:, None] for k in (1, 2, 4, 8)]
        shuf_keep = [iota16 >= k for k in (1, 2, 4, 8)]
        trash = maxc - 16 + iota16
        gd = lax.GatherDimensionNumbers(
            offset_dims=(), collapsed_slice_dims=(0,), start_index_map=(0,))

        def _lane_shift(v, idx):
            return lax.gather(v, idx, gd, (1,),
                              mode=lax.GatherScatterMode.PROMISE_IN_BOUNDS)

        pltpu.sync_copy(src_h.at[pl.ds(base, ew)], srcv)
        pltpu.sync_copy(dst_h.at[pl.ds(base, ew)], dstv)
        pltpu.sync_copy(att_h, attv)

        def _zrow(i, c):
            for j in range(8):
                zbuf[i, pl.ds(16 * j, 16)] = zeros16
            return c
        lax.fori_loop(0, zb, _zrow, 0)

        def _spad(e, c):
            for j in range(8):
                sbuf2[e, pl.ds(16 * j, 16)] = zeros16
            return c
        lax.fori_loop(0, 64, _spad, 0)
        attr = [attv[h] for h in range(8)]

        def _range_body(r, carry):
            lo = r * nr

            def _zc(i, c):
                pltpu.sync_copy(zbuf, acc.at[pl.ds(sid * rpt + i * zb, zb)])
                return c
            lax.fori_loop(0, rpt // zb, _zc, 0)

            def _zc2(i, c):
                pltpu.sync_copy(
                    zbuf, acc2.at[pl.ds(sid * ((nd + 128) // 16) + i * zb, zb)])
                return c
            lax.fori_loop(0, (nd + 128) // 16 // zb, _zc2, 0)
            plsc.subcore_barrier()

            def _cg(g, cnt):
                d = dstv[pl.ds(g * 16, 16)]
                s = srcv[pl.ds(g * 16, 16)]
                dl = d - lo
                msk = (dl >= 0) & (dl < nr)
                pf = jnp.where(msk, 1, 0).astype(_i32)
                for k in range(4):
                    sh = _lane_shift(pf, shuf_idx[k])
                    pf = pf + jnp.where(shuf_keep[k], sh, 0)
                pos = jnp.where(msk, cnt + pf - 1, trash)
                plsc.store_scatter(csrc, [pos], s)
                plsc.store_scatter(cdstl, [pos], dl)
                if has_xe:
                    plsc.store_scatter(ceid, [pos], base + g * 16 + iota16)
                return cnt + pf[15]
            cnt = lax.fori_loop(0, ng, _cg, jnp.array(0, _i32))

            padv = jnp.full((16,), nr, _i32)
            zi = jnp.zeros((16,), _i32)
            for t in range(4):
                csrc[pl.ds(cnt + t * 16, 16)] = zi
                cdstl[pl.ds(cnt + t * 16, 16)] = padv
                if has_xe:
                    ceid[pl.ds(cnt + t * 16, 16)] = zi
            nchunks = (cnt + 63) // 64

            def _ck(k, c):
                for j in range(4):
                    dlj = cdstl[pl.ds(k * 64 + 16 * j, 16)]
                    cdst2[0, pl.ds(16 * j, 16)] = dlj
                    cdst3[0, pl.ds(16 * j, 16)] = dlj >> 3
                    cidx[pl.ds(16 * j, 16)] = dlj + lo
                cp1 = pltpu.async_copy(xl_h.at[csrc.at[pl.ds(k * 64, 64)]],
                                       xlb, sem1)
                cp2 = pltpu.async_copy(xr_h.at[cidx], xrb, sem2)
                if has_xe:
                    cp3 = pltpu.async_copy(xe_h.at[ceid.at[pl.ds(k * 64, 64)]],
                                           xeb, sem3)
                cp1.wait()
                cp2.wait()
                if has_xe:
                    cp3.wait()

                def _ed(e2, cc):
                    for eo in range(2):
                        e = e2 * 2 + eo
                        e16 = jnp.full((16,), e, _i32)
                        dlsp = plsc.load_gather(
                            cdstl, [jnp.full((16,), 0, _i32) + (k * 64 + e)])
                        cb = ((dlsp & 7) << 4)[0]
                        xlr = [xlb[e, pl.ds(16 * j, 16)] for j in range(8)]
                        packed = zeros16
                        for h in range(8):
                            mv = xlr[h] + xrb[e, pl.ds(16 * h, 16)]
                            if has_xe:
                                mv = mv + xeb[e, pl.ds(16 * h, 16)]
                            mav = jnp.maximum(mv, 0.2 * mv)
                            alh = jnp.sum(mav * attr[h])
                            exv = jnp.exp(jnp.full((16,), alh, _f32))
                            sbuf[e, pl.ds(16 * h, 16)] = xlr[h] * exv
                            packed = jnp.where(iota16 == h, exv, packed)
                        sbuf2[e, pl.ds(cb, 16)] = packed
                    return cc
                lax.fori_loop(0, 32, _ed, 0)

                pltpu.sync_copy(sbuf, acc.at[cdst2.at[0]], add=True)
                pltpu.sync_copy(sbuf2, acc2.at[cdst3.at[0]], add=True)

                def _zs(e, cc):
                    dlsp = plsc.load_gather(
                        cdstl, [jnp.full((16,), 0, _i32) + (k * 64 + e)])
                    cb = ((dlsp & 7) << 4)[0]
                    sbuf2[e, pl.ds(cb, 16)] = zeros16
                    return cc
                lax.fori_loop(0, 64, _zs, 0)
                return c
            lax.fori_loop(0, nchunks, _ck, 0)

            plsc.subcore_barrier()

            row0 = sid * (nr // 16)
            pltpu.sync_copy(acc.at[pl.ds(row0, nr // 16)],
                            outp_h.at[cid, pl.ds(lo + row0, nr // 16)])
            row0d = sid * (nd // 16)
            pltpu.sync_copy(acc2.at[pl.ds(row0d, nd // 16)],
                            denp_h.at[cid, pl.ds(r * nd + row0d, nd // 16)])
            plsc.subcore_barrier()
            return carry
        lax.fori_loop(0, r_cnt, _range_body, 0)

    out_type = [
        jax.ShapeDtypeStruct((2, npad, 128), _f32),
        jax.ShapeDtypeStruct((2, npad // 8, 128), _f32),
    ]
    args = [src, dst, xl, xr]
    if has_xe:
        args.append(xe)
    args.append(att2)
    return pl.kernel(
        body, out_type=out_type, mesh=mesh, scratch_types=scratch,
        compiler_params=pltpu.CompilerParams(needs_layout_passes=False),
    )(*args)


def _pad_edges(ei, n, ep):
    e = ei.shape[1]
    src = jnp.concatenate([ei[0], jnp.zeros((ep - e,), _i32)])
    dst = jnp.concatenate([ei[1], jnp.full((ep - e,), n, _i32)])
    return src, dst


def _gat_layer(p, y, src, dst, ep, n, nr, r_cnt, b8, xe=None, fill=None):
    xl, xr = _mm2(y, p)
    att2 = p['att'].reshape(_H, 16)
    attb = jnp.zeros((_D, _H), _f32).at[
        jnp.arange(_D), jnp.arange(_D) // 16].set(p['att'].reshape(_D))
    outp, denp = _edge_call(src, dst, xl, xr, xe, att2, ep, n, nr, r_cnt)
    denr = denp.reshape(2, nr * r_cnt, 16)
    if fill is None:
        fill = jnp.zeros((1, _D), _f32)
    return _combine(outp[0, :n], outp[1, :n], denr[0, :n], denr[1, :n],
                    xl, xr, fill, attb, b8, p, p['_ln'])


def kernel(x_words, x_sent, w2w_index, w2s_index, s2s_index, s2s_type,
           s2s_sim, params):
    b8 = (jnp.arange(_D)[None, :] // 16 ==
          jnp.arange(_H)[:, None]).astype(_f32)

    ew_p = ((w2w_index.shape[1] + 511) // 512) * 512
    ews_p = ((w2s_index.shape[1] + 511) // 512) * 512
    es_p = ((s2s_index.shape[1] + 511) // 512) * 512

    w2w_s, w2w_d = _pad_edges(w2w_index, _NW, ew_p)
    w2s_s, w2s_d = _pad_edges(w2s_index, _NW + _NS, ews_p)
    s2s_s, s2s_d = _pad_edges(s2s_index, _NS, es_p)
    sim_s, sim_d = _pad_edges(s2s_sim, _NS, es_p)

    es = s2s_index.shape[1]
    ea_pad = jnp.concatenate(
        [s2s_type, jnp.zeros((es_p - es, 17), _f32)], axis=0)

    def lp(name, ln):
        q = dict(params[name])
        q['_ln'] = params[ln]
        return q

    y = _gat_layer(lp('w2w_1', 'ln1'), x_words, w2w_s, w2w_d,
                   ew_p, _NW, 4096, 13, b8)
    y = _gat_layer(lp('w2w_1', 'ln2'), y, w2w_s, w2w_d,
                   ew_p, _NW, 4096, 13, b8)
    yc = jnp.concatenate([y, x_sent], axis=0)
    yc = _gat_layer(lp('word_to_sent', 'ln3'), yc, w2s_s, w2s_d,
                    ews_p, _NW + _NS, 4096, 15, b8)
    ys = yc[_NW:]

    xe1, cs1 = _xe(ea_pad, params['s2s_1']['We'])
    ys = _gat_layer(lp('s2s_1', 'ln4'), ys, s2s_s, s2s_d,
                    es_p, _NS, 4096, 3, b8, xe=xe1, fill=cs1 / es)
    xe2, cs2 = _xe(ea_pad, params['s2s_2']['We'])
    ys = _gat_layer(lp('s2s_2', 'ln5'), ys, s2s_s, s2s_d,
                    es_p, _NS, 4096, 3, b8, xe=xe2, fill=cs2 / es)

    ys = _gat_layer(lp('red_1', 'ln6'), ys, sim_s, sim_d,
                    es_p, _NS, 4096, 3, b8)
    ys = _gat_layer(lp('red_2', 'ln7'), ys, sim_s, sim_d,
                    es_p, _NS, 4096, 3, b8)

    return _cls(ys, params)


# speedup vs baseline: 43.9627x; 1.0402x over previous
import functools

import jax
import jax.numpy as jnp
from jax import lax
from jax.experimental import pallas as pl
from jax.experimental.pallas import tpu as pltpu
from jax.experimental.pallas import tpu_sc as plsc

_NW = 50000
_NS = 10000
_D = 128
_H = 8
_OUT = 4

_f32 = jnp.float32
_i32 = jnp.int32


def _mm2_body(y, wl, bl, wr, br, xl, xr):
    yv = y[...]
    xl[...] = jnp.dot(yv, wl[...], preferred_element_type=_f32) + bl[...]
    xr[...] = jnp.dot(yv, wr[...], preferred_element_type=_f32) + br[...]


def _mm2(y, p):
    n = y.shape[0]
    bn = 1000
    return pl.pallas_call(
        _mm2_body,
        grid=(n // bn,),
        in_specs=[
            pl.BlockSpec((bn, _D), lambda i: (i, 0)),
            pl.BlockSpec((_D, _D), lambda i: (0, 0)),
            pl.BlockSpec((1, _D), lambda i: (0, 0)),
            pl.BlockSpec((_D, _D), lambda i: (0, 0)),
            pl.BlockSpec((1, _D), lambda i: (0, 0)),
        ],
        out_specs=[pl.BlockSpec((bn, _D), lambda i: (i, 0))] * 2,
        out_shape=[jax.ShapeDtypeStruct((n, _D), _f32)] * 2,
    )(y, p['Wl'], p['bl'][None], p['Wr'], p['br'][None])


def _xe_body(ea, we, xe, cs):
    x = jnp.dot(ea[...], we[...], preferred_element_type=_f32)
    xe[...] = x

    @pl.when(pl.program_id(0) == 0)
    def _():
        cs[...] = jnp.zeros_like(cs)

    cs[...] += jnp.sum(x, axis=0, keepdims=True)


def _xe(ea_pad, we):
    ep = ea_pad.shape[0]
    be = 512
    return pl.pallas_call(
        _xe_body,
        grid=(ep // be,),
        in_specs=[
            pl.BlockSpec((be, 17), lambda i: (i, 0)),
            pl.BlockSpec((17, _D), lambda i: (0, 0)),
        ],
        out_specs=[
            pl.BlockSpec((be, _D), lambda i: (i, 0)),
            pl.BlockSpec((1, _D), lambda i: (0, 0)),
        ],
        out_shape=[
            jax.ShapeDtypeStruct((ep, _D), _f32),
            jax.ShapeDtypeStruct((1, _D), _f32),
        ],
    )(ea_pad, we)


def _combine_body(p0, p1, d0, d1, xl, xr, fill, attb, b8, bias, g, b, out):
    xlv = xl[...]
    m = xlv + xr[...] + fill[...]
    ma = jnp.maximum(m, 0.2 * m)
    exs = jnp.exp(jnp.dot(ma, attb[...], preferred_element_type=_f32))
    den = d0[...][:, :8] + d1[...][:, :8] + exs
    acc = p0[...] + p1[...] + xlv * jnp.dot(exs, b8[...], preferred_element_type=_f32)
    o = acc / (jnp.dot(den, b8[...], preferred_element_type=_f32) + 1e-16)
    o = o + bias[...]
    mu = jnp.mean(o, axis=1, keepdims=True)
    var = jnp.mean(jnp.square(o - mu), axis=1, keepdims=True)
    o = (o - mu) * lax.rsqrt(var + 1e-5) * g[...] + b[...]
    out[...] = jnp.maximum(o, 0.0)


def _combine(p0, p1, d0, d1, xl, xr, fill, attb, b8, p, lnp):
    n = xl.shape[0]
    bn = 1000
    return pl.pallas_call(
        _combine_body,
        grid=(n // bn,),
        in_specs=[
            pl.BlockSpec((bn, _D), lambda i: (i, 0)),
            pl.BlockSpec((bn, _D), lambda i: (i, 0)),
            pl.BlockSpec((bn, 16), lambda i: (i, 0)),
            pl.BlockSpec((bn, 16), lambda i: (i, 0)),
            pl.BlockSpec((bn, _D), lambda i: (i, 0)),
            pl.BlockSpec((bn, _D), lambda i: (i, 0)),
            pl.BlockSpec((1, _D), lambda i: (0, 0)),
            pl.BlockSpec((_D, _H), lambda i: (0, 0)),
            pl.BlockSpec((_H, _D), lambda i: (0, 0)),
            pl.BlockSpec((1, _D), lambda i: (0, 0)),
            pl.BlockSpec((1, _D), lambda i: (0, 0)),
            pl.BlockSpec((1, _D), lambda i: (0, 0)),
        ],
        out_specs=pl.BlockSpec((bn, _D), lambda i: (i, 0)),
        out_shape=jax.ShapeDtypeStruct((n, _D), _f32),
    )(p0, p1, d0, d1, xl, xr, fill, attb, b8, p['bias'][None],
      lnp['g'][None], lnp['b'][None])


def _cls_body(y, w1, b1, g, b, w2, b2, out):
    h = jnp.dot(y[...], w1[...], preferred_element_type=_f32) + b1[...]
    mu = jnp.mean(h, axis=1, keepdims=True)
    var = jnp.mean(jnp.square(h - mu), axis=1, keepdims=True)
    h = (h - mu) * lax.rsqrt(var + 1e-5) * g[...] + b[...]
    h = jnp.maximum(h, 0.0)
    out[...] = jnp.dot(h, w2[...], preferred_element_type=_f32) + b2[...]


def _cls(y, params):
    n = y.shape[0]
    bn = 1000
    dq = _D // 4
    return pl.pallas_call(
        _cls_body,
        grid=(n // bn,),
        in_specs=[
            pl.BlockSpec((bn, _D), lambda i: (i, 0)),
            pl.BlockSpec((_D, dq), lambda i: (0, 0)),
            pl.BlockSpec((1, dq), lambda i: (0, 0)),
            pl.BlockSpec((1, dq), lambda i: (0, 0)),
            pl.BlockSpec((1, dq), lambda i: (0, 0)),
            pl.BlockSpec((dq, _OUT), lambda i: (0, 0)),
            pl.BlockSpec((1, _OUT), lambda i: (0, 0)),
        ],
        out_specs=pl.BlockSpec((bn, _OUT), lambda i: (i, 0)),
        out_shape=jax.ShapeDtypeStruct((n, _OUT), _f32),
    )(y, params['cls_W1'], params['cls_b1'][None], params['cls_ln']['g'][None],
      params['cls_ln']['b'][None], params['cls_W2'], params['cls_b2'][None])


def _edge_call(src, dst, xl, xr, xe, att2, ep, n, nr, r_cnt):
    has_xe = xe is not None
    ew = ep // 32
    ng = ew // 16
    maxc = ((ew + 63) // 64) * 64 + 128
    nrp = nr + 512
    rpt = nrp // 16
    zb = 8
    nd = nr // 8
    npad = nr * r_cnt

    mesh = plsc.VectorSubcoreMesh(core_axis_name="c", subcore_axis_name="s")

    scratch = [
        pltpu.VMEM((ew,), _i32),
        pltpu.VMEM((ew,), _i32),
        pltpu.VMEM((maxc,), _i32),
        pltpu.VMEM((maxc,), _i32),
        pltpu.VMEM((64,), _i32),
        pltpu.VMEM((1, 64), _i32),
        pltpu.VMEM((1, 64), _i32),
        pltpu.VMEM((64, _D), _f32),
        pltpu.VMEM((64, _D), _f32),
        pltpu.VMEM((64, _D), _f32),
        pltpu.VMEM((64, _D), _f32),
        pltpu.VMEM((_H, 16), _f32),
        pltpu.VMEM((zb, _D), _f32),
        pltpu.VMEM_SHARED((nrp, _D), _f32),
        pltpu.VMEM_SHARED((nr // 8 + 128, _D), _f32),
        pltpu.SemaphoreType.DMA,
        pltpu.SemaphoreType.DMA,
        pltpu.SemaphoreType.DMA,
    ]
    if has_xe:
        scratch = scratch[:7] + [pltpu.VMEM((maxc,), _i32),
                                 pltpu.VMEM((64, _D), _f32)] + scratch[7:]

    def body(*refs):
        if has_xe:
            (src_h, dst_h, xl_h, xr_h, xe_h, att_h, outp_h, denp_h,
             srcv, dstv, csrc, cdstl, cidx, cdst2, cdst3, ceid, xeb,
             xlb, xrb, sbuf, sbuf2, attv, zbuf, acc, acc2,
             sem1, sem2, sem3) = refs
        else:
            (src_h, dst_h, xl_h, xr_h, att_h, outp_h, denp_h,
             srcv, dstv, csrc, cdstl, cidx, cdst2, cdst3,
             xlb, xrb, sbuf, sbuf2, attv, zbuf, acc, acc2,
             sem1, sem2, sem3) = refs

        cid = lax.axis_index("c")
        sid = lax.axis_index("s")
        wid = sid * 2 + cid
        base = wid * ew
        iota16 = lax.iota(_i32, 16)
        zeros16 = jnp.zeros((16,), _f32)
        shuf_idx = [jnp.maximum(iota16 - k, 0)[:, None] for k in (1, 2, 4, 8)]
        shuf_keep = [iota16 >= k for k in (1, 2, 4, 8)]
        trash = maxc - 16 + iota16
        gd = lax.GatherDimensionNumbers(
            offset_dims=(), collapsed_slice_dims=(0,), start_index_map=(0,))

        def _lane_shift(v, idx):
            return lax.gather(v, idx, gd, (1,),
                              mode=lax.GatherScatterMode.PROMISE_IN_BOUNDS)

        pltpu.sync_copy(src_h.at[pl.ds(base, ew)], srcv)
        pltpu.sync_copy(dst_h.at[pl.ds(base, ew)], dstv)
        pltpu.sync_copy(att_h, attv)

        def _zrow(i, c):
            for j in range(8):
                zbuf[i, pl.ds(16 * j, 16)] = zeros16
            return c
        lax.fori_loop(0, zb, _zrow, 0)

        def _spad(e, c):
            for j in range(8):
                sbuf2[e, pl.ds(16 * j, 16)] = zeros16
            return c
        lax.fori_loop(0, 64, _spad, 0)
        attr = [attv[h] for h in range(8)]

        def _range_body(r, carry):
            lo = r * nr

            def _zc(i, c):
                pltpu.sync_copy(zbuf, acc.at[pl.ds(sid * rpt + i * zb, zb)])
                return c
            lax.fori_loop(0, rpt // zb, _zc, 0)

            def _zc2(i, c):
                pltpu.sync_copy(
                    zbuf, acc2.at[pl.ds(sid * ((nd + 128) // 16) + i * zb, zb)])
                return c
            lax.fori_loop(0, (nd + 128) // 16 // zb, _zc2, 0)
            plsc.subcore_barrier()

            def _cg(g, cnt):
                d = dstv[pl.ds(g * 16, 16)]
                s = srcv[pl.ds(g * 16, 16)]
                dl = d - lo
                msk = (dl >= 0) & (dl < nr)
                pf = jnp.where(msk, 1, 0).astype(_i32)
                for k in range(4):
                    sh = _lane_shift(pf, shuf_idx[k])
                    pf = pf + jnp.where(shuf_keep[k], sh, 0)
                pos = jnp.where(msk, cnt + pf - 1, trash)
                plsc.store_scatter(csrc, [pos], s)
                plsc.store_scatter(cdstl, [pos], dl)
                if has_xe:
                    plsc.store_scatter(ceid, [pos], base + g * 16 + iota16)
                return cnt + pf[15]
            cnt = lax.fori_loop(0, ng, _cg, jnp.array(0, _i32))

            padv = jnp.full((16,), nr, _i32)
            zi = jnp.zeros((16,), _i32)
            for t in range(4):
                csrc[pl.ds(cnt + t * 16, 16)] = zi
                cdstl[pl.ds(cnt + t * 16, 16)] = padv
                if has_xe:
                    ceid[pl.ds(cnt + t * 16, 16)] = zi
            nchunks = (cnt + 63) // 64

            def _ck(k, c):
                for j in range(4):
                    dlj = cdstl[pl.ds(k * 64 + 16 * j, 16)]
                    cdst2[0, pl.ds(16 * j, 16)] = dlj
                    cdst3[0, pl.ds(16 * j, 16)] = dlj >> 3
                    cidx[pl.ds(16 * j, 16)] = dlj + lo
                cp1 = pltpu.async_copy(xl_h.at[csrc.at[pl.ds(k * 64, 64)]],
                                       xlb, sem1)
                cp2 = pltpu.async_copy(xr_h.at[cidx], xrb, sem2)
                if has_xe:
                    cp3 = pltpu.async_copy(xe_h.at[ceid.at[pl.ds(k * 64, 64)]],
                                           xeb, sem3)
                cp1.wait()
                cp2.wait()
                if has_xe:
                    cp3.wait()

                def _ed(e2, cc):
                    es = [e2 * 2, e2 * 2 + 1]
                    cbs = []
                    for e in es:
                        dlsp = plsc.load_gather(
                            cdstl, [jnp.full((16,), 0, _i32) + (k * 64 + e)])
                        cbs.append(((dlsp & 7) << 4)[0])
                    xls = [[xlb[e, pl.ds(16 * j, 16)] for j in range(8)]
                           for e in es]
                    ts = []
                    for i, e in enumerate(es):
                        for h in range(8):
                            mv = xls[i][h] + xrb[e, pl.ds(16 * h, 16)]
                            if has_xe:
                                mv = mv + xeb[e, pl.ds(16 * h, 16)]
                            ts.append(jnp.maximum(mv, 0.2 * mv) * attr[h])
                    als = [jnp.sum(t) for t in ts]
                    exvs = [jnp.exp(jnp.full((16,), a, _f32)) for a in als]
                    for i, e in enumerate(es):
                        packed = zeros16
                        for h in range(8):
                            exv = exvs[i * 8 + h]
                            sbuf[e, pl.ds(16 * h, 16)] = xls[i][h] * exv
                            packed = jnp.where(iota16 == h, exv, packed)
                        sbuf2[e, pl.ds(cbs[i], 16)] = packed
                    return cc
                lax.fori_loop(0, 32, _ed, 0)

                pltpu.sync_copy(sbuf, acc.at[cdst2.at[0]], add=True)
                pltpu.sync_copy(sbuf2, acc2.at[cdst3.at[0]], add=True)

                def _zs(e, cc):
                    dlsp = plsc.load_gather(
                        cdstl, [jnp.full((16,), 0, _i32) + (k * 64 + e)])
                    cb = ((dlsp & 7) << 4)[0]
                    sbuf2[e, pl.ds(cb, 16)] = zeros16
                    return cc
                lax.fori_loop(0, 64, _zs, 0)
                return c
            lax.fori_loop(0, nchunks, _ck, 0)

            plsc.subcore_barrier()

            row0 = sid * (nr // 16)
            pltpu.sync_copy(acc.at[pl.ds(row0, nr // 16)],
                            outp_h.at[cid, pl.ds(lo + row0, nr // 16)])
            row0d = sid * (nd // 16)
            pltpu.sync_copy(acc2.at[pl.ds(row0d, nd // 16)],
                            denp_h.at[cid, pl.ds(r * nd + row0d, nd // 16)])
            plsc.subcore_barrier()
            return carry
        lax.fori_loop(0, r_cnt, _range_body, 0)

    out_type = [
        jax.ShapeDtypeStruct((2, npad, 128), _f32),
        jax.ShapeDtypeStruct((2, npad // 8, 128), _f32),
    ]
    args = [src, dst, xl, xr]
    if has_xe:
        args.append(xe)
    args.append(att2)
    return pl.kernel(
        body, out_type=out_type, mesh=mesh, scratch_types=scratch,
        compiler_params=pltpu.CompilerParams(needs_layout_passes=False),
    )(*args)


def _pad_edges(ei, n, ep):
    e = ei.shape[1]
    src = jnp.concatenate([ei[0], jnp.zeros((ep - e,), _i32)])
    dst = jnp.concatenate([ei[1], jnp.full((ep - e,), n, _i32)])
    return src, dst


def _gat_layer(p, y, src, dst, ep, n, nr, r_cnt, b8, xe=None, fill=None):
    xl, xr = _mm2(y, p)
    att2 = p['att'].reshape(_H, 16)
    attb = jnp.zeros((_D, _H), _f32).at[
        jnp.arange(_D), jnp.arange(_D) // 16].set(p['att'].reshape(_D))
    outp, denp = _edge_call(src, dst, xl, xr, xe, att2, ep, n, nr, r_cnt)
    denr = denp.reshape(2, nr * r_cnt, 16)
    if fill is None:
        fill = jnp.zeros((1, _D), _f32)
    return _combine(outp[0, :n], outp[1, :n], denr[0, :n], denr[1, :n],
                    xl, xr, fill, attb, b8, p, p['_ln'])


def kernel(x_words, x_sent, w2w_index, w2s_index, s2s_index, s2s_type,
           s2s_sim, params):
    b8 = (jnp.arange(_D)[None, :] // 16 ==
          jnp.arange(_H)[:, None]).astype(_f32)

    ew_p = ((w2w_index.shape[1] + 511) // 512) * 512
    ews_p = ((w2s_index.shape[1] + 511) // 512) * 512
    es_p = ((s2s_index.shape[1] + 511) // 512) * 512

    w2w_s, w2w_d = _pad_edges(w2w_index, _NW, ew_p)
    w2s_s, w2s_d = _pad_edges(w2s_index, _NW + _NS, ews_p)
    s2s_s, s2s_d = _pad_edges(s2s_index, _NS, es_p)
    sim_s, sim_d = _pad_edges(s2s_sim, _NS, es_p)

    es = s2s_index.shape[1]
    ea_pad = jnp.concatenate(
        [s2s_type, jnp.zeros((es_p - es, 17), _f32)], axis=0)

    def lp(name, ln):
        q = dict(params[name])
        q['_ln'] = params[ln]
        return q

    y = _gat_layer(lp('w2w_1', 'ln1'), x_words, w2w_s, w2w_d,
                   ew_p, _NW, 4096, 13, b8)
    y = _gat_layer(lp('w2w_1', 'ln2'), y, w2w_s, w2w_d,
                   ew_p, _NW, 4096, 13, b8)
    yc = jnp.concatenate([y, x_sent], axis=0)
    yc = _gat_layer(lp('word_to_sent', 'ln3'), yc, w2s_s, w2s_d,
                    ews_p, _NW + _NS, 4096, 15, b8)
    ys = yc[_NW:]

    xe1, cs1 = _xe(ea_pad, params['s2s_1']['We'])
    ys = _gat_layer(lp('s2s_1', 'ln4'), ys, s2s_s, s2s_d,
                    es_p, _NS, 4096, 3, b8, xe=xe1, fill=cs1 / es)
    xe2, cs2 = _xe(ea_pad, params['s2s_2']['We'])
    ys = _gat_layer(lp('s2s_2', 'ln5'), ys, s2s_s, s2s_d,
                    es_p, _NS, 4096, 3, b8, xe=xe2, fill=cs2 / es)

    ys = _gat_layer(lp('red_1', 'ln6'), ys, sim_s, sim_d,
                    es_p, _NS, 4096, 3, b8)
    ys = _gat_layer(lp('red_2', 'ln7'), ys, sim_s, sim_d,
                    es_p, _NS, 4096, 3, b8)

    return _cls(ys, params)


# speedup vs baseline: 44.8702x; 1.0206x over previous
import functools

import jax
import jax.numpy as jnp
from jax import lax
from jax.experimental import pallas as pl
from jax.experimental.pallas import tpu as pltpu
from jax.experimental.pallas import tpu_sc as plsc

_NW = 50000
_NS = 10000
_D = 128
_H = 8
_OUT = 4

_f32 = jnp.float32
_i32 = jnp.int32


def _mm2_body(y, wl, bl, wr, br, xl, xr):
    yv = y[...]
    xl[...] = jnp.dot(yv, wl[...], preferred_element_type=_f32) + bl[...]
    xr[...] = jnp.dot(yv, wr[...], preferred_element_type=_f32) + br[...]


def _mm2(y, p):
    n = y.shape[0]
    bn = 1000
    return pl.pallas_call(
        _mm2_body,
        grid=(n // bn,),
        in_specs=[
            pl.BlockSpec((bn, _D), lambda i: (i, 0)),
            pl.BlockSpec((_D, _D), lambda i: (0, 0)),
            pl.BlockSpec((1, _D), lambda i: (0, 0)),
            pl.BlockSpec((_D, _D), lambda i: (0, 0)),
            pl.BlockSpec((1, _D), lambda i: (0, 0)),
        ],
        out_specs=[pl.BlockSpec((bn, _D), lambda i: (i, 0))] * 2,
        out_shape=[jax.ShapeDtypeStruct((n, _D), _f32)] * 2,
    )(y, p['Wl'], p['bl'][None], p['Wr'], p['br'][None])


def _xe_body(ea, we, xe, cs):
    x = jnp.dot(ea[...], we[...], preferred_element_type=_f32)
    xe[...] = x

    @pl.when(pl.program_id(0) == 0)
    def _():
        cs[...] = jnp.zeros_like(cs)

    cs[...] += jnp.sum(x, axis=0, keepdims=True)


def _xe(ea_pad, we):
    ep = ea_pad.shape[0]
    be = 512
    return pl.pallas_call(
        _xe_body,
        grid=(ep // be,),
        in_specs=[
            pl.BlockSpec((be, 17), lambda i: (i, 0)),
            pl.BlockSpec((17, _D), lambda i: (0, 0)),
        ],
        out_specs=[
            pl.BlockSpec((be, _D), lambda i: (i, 0)),
            pl.BlockSpec((1, _D), lambda i: (0, 0)),
        ],
        out_shape=[
            jax.ShapeDtypeStruct((ep, _D), _f32),
            jax.ShapeDtypeStruct((1, _D), _f32),
        ],
    )(ea_pad, we)


def _combine_body(p0, p1, d0, d1, xl, xr, fill, attb, b8, bias, g, b, out):
    xlv = xl[...]
    m = xlv + xr[...] + fill[...]
    ma = jnp.maximum(m, 0.2 * m)
    exs = jnp.exp(jnp.dot(ma, attb[...], preferred_element_type=_f32))
    den = d0[...][:, :8] + d1[...][:, :8] + exs
    acc = p0[...] + p1[...] + xlv * jnp.dot(exs, b8[...], preferred_element_type=_f32)
    o = acc / (jnp.dot(den, b8[...], preferred_element_type=_f32) + 1e-16)
    o = o + bias[...]
    mu = jnp.mean(o, axis=1, keepdims=True)
    var = jnp.mean(jnp.square(o - mu), axis=1, keepdims=True)
    o = (o - mu) * lax.rsqrt(var + 1e-5) * g[...] + b[...]
    out[...] = jnp.maximum(o, 0.0)


def _combine(p0, p1, d0, d1, xl, xr, fill, attb, b8, p, lnp):
    n = xl.shape[0]
    bn = 1000
    return pl.pallas_call(
        _combine_body,
        grid=(n // bn,),
        in_specs=[
            pl.BlockSpec((bn, _D), lambda i: (i, 0)),
            pl.BlockSpec((bn, _D), lambda i: (i, 0)),
            pl.BlockSpec((bn, 16), lambda i: (i, 0)),
            pl.BlockSpec((bn, 16), lambda i: (i, 0)),
            pl.BlockSpec((bn, _D), lambda i: (i, 0)),
            pl.BlockSpec((bn, _D), lambda i: (i, 0)),
            pl.BlockSpec((1, _D), lambda i: (0, 0)),
            pl.BlockSpec((_D, _H), lambda i: (0, 0)),
            pl.BlockSpec((_H, _D), lambda i: (0, 0)),
            pl.BlockSpec((1, _D), lambda i: (0, 0)),
            pl.BlockSpec((1, _D), lambda i: (0, 0)),
            pl.BlockSpec((1, _D), lambda i: (0, 0)),
        ],
        out_specs=pl.BlockSpec((bn, _D), lambda i: (i, 0)),
        out_shape=jax.ShapeDtypeStruct((n, _D), _f32),
    )(p0, p1, d0, d1, xl, xr, fill, attb, b8, p['bias'][None],
      lnp['g'][None], lnp['b'][None])


def _cls_body(y, w1, b1, g, b, w2, b2, out):
    h = jnp.dot(y[...], w1[...], preferred_element_type=_f32) + b1[...]
    mu = jnp.mean(h, axis=1, keepdims=True)
    var = jnp.mean(jnp.square(h - mu), axis=1, keepdims=True)
    h = (h - mu) * lax.rsqrt(var + 1e-5) * g[...] + b[...]
    h = jnp.maximum(h, 0.0)
    out[...] = jnp.dot(h, w2[...], preferred_element_type=_f32) + b2[...]


def _cls(y, params):
    n = y.shape[0]
    bn = 1000
    dq = _D // 4
    return pl.pallas_call(
        _cls_body,
        grid=(n // bn,),
        in_specs=[
            pl.BlockSpec((bn, _D), lambda i: (i, 0)),
            pl.BlockSpec((_D, dq), lambda i: (0, 0)),
            pl.BlockSpec((1, dq), lambda i: (0, 0)),
            pl.BlockSpec((1, dq), lambda i: (0, 0)),
            pl.BlockSpec((1, dq), lambda i: (0, 0)),
            pl.BlockSpec((dq, _OUT), lambda i: (0, 0)),
            pl.BlockSpec((1, _OUT), lambda i: (0, 0)),
        ],
        out_specs=pl.BlockSpec((bn, _OUT), lambda i: (i, 0)),
        out_shape=jax.ShapeDtypeStruct((n, _OUT), _f32),
    )(y, params['cls_W1'], params['cls_b1'][None], params['cls_ln']['g'][None],
      params['cls_ln']['b'][None], params['cls_W2'], params['cls_b2'][None])


def _edge_call(src, dst, xl, xr, xe, att2, ep, n, nr, r_cnt):
    has_xe = xe is not None
    ew = ep // 32
    ng = ew // 16
    maxc = ((ew + 63) // 64) * 64 + 128
    nrp = nr + 512
    rpt = nrp // 16
    zb = 8
    nd = nr // 8
    npad = nr * r_cnt

    mesh = plsc.VectorSubcoreMesh(core_axis_name="c", subcore_axis_name="s")

    scratch = [
        pltpu.VMEM((ew,), _i32),
        pltpu.VMEM((ew,), _i32),
        pltpu.VMEM((maxc,), _i32),
        pltpu.VMEM((64,), _i32),
        pltpu.VMEM((64,), _i32),
        pltpu.VMEM((1, 64), _i32),
        pltpu.VMEM((1, 64), _i32),
        pltpu.VMEM((64, _D), _f32),
        pltpu.VMEM((64, _D), _f32),
        pltpu.VMEM((64, _D), _f32),
        pltpu.VMEM((64, _D), _f32),
        pltpu.VMEM((_H, 16), _f32),
        pltpu.VMEM((zb, _D), _f32),
        pltpu.VMEM_SHARED((nrp, _D), _f32),
        pltpu.VMEM_SHARED((nr // 8 + 128, _D), _f32),
        pltpu.SemaphoreType.DMA,
        pltpu.SemaphoreType.DMA,
        pltpu.SemaphoreType.DMA,
    ]
    if has_xe:
        scratch = scratch[:7] + [pltpu.VMEM((64,), _i32),
                                 pltpu.VMEM((64, _D), _f32)] + scratch[7:]

    def body(*refs):
        if has_xe:
            (src_h, dst_h, xl_h, xr_h, xe_h, att_h, outp_h, denp_h,
             srcv, dstv, ceid, cidxs, cidx, cdst2, cdst3, cidxe, xeb,
             xlb, xrb, sbuf, sbuf2, attv, zbuf, acc, acc2,
             sem1, sem2, sem3) = refs
        else:
            (src_h, dst_h, xl_h, xr_h, att_h, outp_h, denp_h,
             srcv, dstv, ceid, cidxs, cidx, cdst2, cdst3,
             xlb, xrb, sbuf, sbuf2, attv, zbuf, acc, acc2,
             sem1, sem2, sem3) = refs

        cid = lax.axis_index("c")
        sid = lax.axis_index("s")
        wid = sid * 2 + cid
        base = wid * ew
        iota16 = lax.iota(_i32, 16)
        zeros16 = jnp.zeros((16,), _f32)
        shuf_idx = [jnp.maximum(iota16 - k, 0)[:, None] for k in (1, 2, 4, 8)]
        shuf_keep = [iota16 >= k for k in (1, 2, 4, 8)]
        trash = maxc - 16 + iota16
        gd = lax.GatherDimensionNumbers(
            offset_dims=(), collapsed_slice_dims=(0,), start_index_map=(0,))

        def _lane_shift(v, idx):
            return lax.gather(v, idx, gd, (1,),
                              mode=lax.GatherScatterMode.PROMISE_IN_BOUNDS)

        pltpu.sync_copy(src_h.at[pl.ds(base, ew)], srcv)
        pltpu.sync_copy(dst_h.at[pl.ds(base, ew)], dstv)
        pltpu.sync_copy(att_h, attv)

        def _zrow(i, c):
            for j in range(8):
                zbuf[i, pl.ds(16 * j, 16)] = zeros16
            return c
        lax.fori_loop(0, zb, _zrow, 0)

        def _spad(e, c):
            for j in range(8):
                sbuf2[e, pl.ds(16 * j, 16)] = zeros16
            return c
        lax.fori_loop(0, 64, _spad, 0)
        attr = [attv[h] for h in range(8)]

        def _range_body(r, carry):
            lo = r * nr

            def _zc(i, c):
                pltpu.sync_copy(zbuf, acc.at[pl.ds(sid * rpt + i * zb, zb)])
                return c
            lax.fori_loop(0, rpt // zb, _zc, 0)

            def _zc2(i, c):
                pltpu.sync_copy(
                    zbuf, acc2.at[pl.ds(sid * ((nd + 128) // 16) + i * zb, zb)])
                return c
            lax.fori_loop(0, (nd + 128) // 16 // zb, _zc2, 0)
            plsc.subcore_barrier()

            def _cg(g, cnt):
                d = dstv[pl.ds(g * 16, 16)]
                dl = d - lo
                msk = (dl >= 0) & (dl < nr)
                pf = jnp.where(msk, 1, 0).astype(_i32)
                for kk in range(4):
                    sh = _lane_shift(pf, shuf_idx[kk])
                    pf = pf + jnp.where(shuf_keep[kk], sh, 0)
                pos = jnp.where(msk, cnt + pf - 1, trash)
                plsc.store_scatter(ceid, [pos], g * 16 + iota16)
                return cnt + pf[15]
            cnt = lax.fori_loop(0, ng, _cg, jnp.array(0, _i32))
            nchunks = (cnt + 63) // 64

            def _ck(k, c):
                for j in range(4):
                    posv = k * 64 + 16 * j + iota16
                    eidj = jnp.clip(ceid[pl.ds(k * 64 + 16 * j, 16)],
                                    0, ew - 1)
                    srcj = plsc.load_gather(srcv, [eidj])
                    dstj = plsc.load_gather(dstv, [eidj])
                    dlj = jnp.where(posv < cnt, dstj - lo, nr)
                    cidxs[pl.ds(16 * j, 16)] = srcj
                    cdst2[0, pl.ds(16 * j, 16)] = dlj
                    cdst3[0, pl.ds(16 * j, 16)] = dlj >> 3
                    cidx[pl.ds(16 * j, 16)] = jnp.minimum(dlj + lo, n - 1)
                    if has_xe:
                        cidxe[pl.ds(16 * j, 16)] = eidj + base
                cp1 = pltpu.async_copy(xl_h.at[cidxs], xlb, sem1)
                cp2 = pltpu.async_copy(xr_h.at[cidx], xrb, sem2)
                if has_xe:
                    cp3 = pltpu.async_copy(xe_h.at[cidxe], xeb, sem3)
                cp1.wait()
                cp2.wait()
                if has_xe:
                    cp3.wait()

                def _ed(e2, cc):
                    es = [e2 * 2, e2 * 2 + 1]
                    cbs = []
                    for e in es:
                        dlsp = plsc.load_gather(
                            cdst2, [jnp.zeros((16,), _i32),
                                    jnp.full((16,), 0, _i32) + e])
                        cbs.append(((dlsp & 7) << 4)[0])
                    xls = [[xlb[e, pl.ds(16 * j, 16)] for j in range(8)]
                           for e in es]
                    ts = []
                    for i, e in enumerate(es):
                        for h in range(8):
                            mv = xls[i][h] + xrb[e, pl.ds(16 * h, 16)]
                            if has_xe:
                                mv = mv + xeb[e, pl.ds(16 * h, 16)]
                            ts.append(jnp.maximum(mv, 0.2 * mv) * attr[h])
                    als = [jnp.sum(t) for t in ts]
                    exvs = [jnp.exp(jnp.full((16,), a, _f32)) for a in als]
                    for i, e in enumerate(es):
                        packed = zeros16
                        for h in range(8):
                            exv = exvs[i * 8 + h]
                            sbuf[e, pl.ds(16 * h, 16)] = xls[i][h] * exv
                            packed = jnp.where(iota16 == h, exv, packed)
                        sbuf2[e, pl.ds(cbs[i], 16)] = packed
                    return cc
                lax.fori_loop(0, 32, _ed, 0)

                pltpu.sync_copy(sbuf, acc.at[cdst2.at[0]], add=True)
                pltpu.sync_copy(sbuf2, acc2.at[cdst3.at[0]], add=True)

                def _zs(e, cc):
                    dlsp = plsc.load_gather(
                        cdst2, [jnp.zeros((16,), _i32),
                                jnp.full((16,), 0, _i32) + e])
                    cb = ((dlsp & 7) << 4)[0]
                    sbuf2[e, pl.ds(cb, 16)] = zeros16
                    return cc
                lax.fori_loop(0, 64, _zs, 0)
                return c
            lax.fori_loop(0, nchunks, _ck, 0)

            plsc.subcore_barrier()

            row0 = sid * (nr // 16)
            pltpu.sync_copy(acc.at[pl.ds(row0, nr // 16)],
                            outp_h.at[cid, pl.ds(lo + row0, nr // 16)])
            row0d = sid * (nd // 16)
            pltpu.sync_copy(acc2.at[pl.ds(row0d, nd // 16)],
                            denp_h.at[cid, pl.ds(r * nd + row0d, nd // 16)])
            plsc.subcore_barrier()
            return carry
        lax.fori_loop(0, r_cnt, _range_body, 0)

    out_type = [
        jax.ShapeDtypeStruct((2, npad, 128), _f32),
        jax.ShapeDtypeStruct((2, npad // 8, 128), _f32),
    ]
    args = [src, dst, xl, xr]
    if has_xe:
        args.append(xe)
    args.append(att2)
    return pl.kernel(
        body, out_type=out_type, mesh=mesh, scratch_types=scratch,
        compiler_params=pltpu.CompilerParams(needs_layout_passes=False),
    )(*args)


def _pad_edges(ei, n, ep):
    e = ei.shape[1]
    src = jnp.concatenate([ei[0], jnp.zeros((ep - e,), _i32)])
    dst = jnp.concatenate([ei[1], jnp.full((ep - e,), n, _i32)])
    return src, dst


def _gat_layer(p, y, src, dst, ep, n, nr, r_cnt, b8, xe=None, fill=None):
    xl, xr = _mm2(y, p)
    att2 = p['att'].reshape(_H, 16)
    attb = jnp.zeros((_D, _H), _f32).at[
        jnp.arange(_D), jnp.arange(_D) // 16].set(p['att'].reshape(_D))
    outp, denp = _edge_call(src, dst, xl, xr, xe, att2, ep, n, nr, r_cnt)
    denr = denp.reshape(2, nr * r_cnt, 16)
    if fill is None:
        fill = jnp.zeros((1, _D), _f32)
    return _combine(outp[0, :n], outp[1, :n], denr[0, :n], denr[1, :n],
                    xl, xr, fill, attb, b8, p, p['_ln'])


def kernel(x_words, x_sent, w2w_index, w2s_index, s2s_index, s2s_type,
           s2s_sim, params):
    b8 = (jnp.arange(_D)[None, :] // 16 ==
          jnp.arange(_H)[:, None]).astype(_f32)

    ew_p = ((w2w_index.shape[1] + 511) // 512) * 512
    ews_p = ((w2s_index.shape[1] + 511) // 512) * 512
    es_p = ((s2s_index.shape[1] + 511) // 512) * 512

    w2w_s, w2w_d = _pad_edges(w2w_index, _NW, ew_p)
    w2s_s, w2s_d = _pad_edges(w2s_index, _NW + _NS, ews_p)
    s2s_s, s2s_d = _pad_edges(s2s_index, _NS, es_p)
    sim_s, sim_d = _pad_edges(s2s_sim, _NS, es_p)

    es = s2s_index.shape[1]
    ea_pad = jnp.concatenate(
        [s2s_type, jnp.zeros((es_p - es, 17), _f32)], axis=0)

    def lp(name, ln):
        q = dict(params[name])
        q['_ln'] = params[ln]
        return q

    y = _gat_layer(lp('w2w_1', 'ln1'), x_words, w2w_s, w2w_d,
                   ew_p, _NW, 4096, 13, b8)
    y = _gat_layer(lp('w2w_1', 'ln2'), y, w2w_s, w2w_d,
                   ew_p, _NW, 4096, 13, b8)
    yc = jnp.concatenate([y, x_sent], axis=0)
    yc = _gat_layer(lp('word_to_sent', 'ln3'), yc, w2s_s, w2s_d,
                    ews_p, _NW + _NS, 4096, 15, b8)
    ys = yc[_NW:]

    xe1, cs1 = _xe(ea_pad, params['s2s_1']['We'])
    ys = _gat_layer(lp('s2s_1', 'ln4'), ys, s2s_s, s2s_d,
                    es_p, _NS, 4096, 3, b8, xe=xe1, fill=cs1 / es)
    xe2, cs2 = _xe(ea_pad, params['s2s_2']['We'])
    ys = _gat_layer(lp('s2s_2', 'ln5'), ys, s2s_s, s2s_d,
                    es_p, _NS, 4096, 3, b8, xe=xe2, fill=cs2 / es)

    ys = _gat_layer(lp('red_1', 'ln6'), ys, sim_s, sim_d,
                    es_p, _NS, 4096, 3, b8)
    ys = _gat_layer(lp('red_2', 'ln7'), ys, sim_s, sim_d,
                    es_p, _NS, 4096, 3, b8)

    return _cls(ys, params)


# speedup vs baseline: 45.9201x; 1.0234x over previous
import functools

import jax
import jax.numpy as jnp
from jax import lax
from jax.experimental import pallas as pl
from jax.experimental.pallas import tpu as pltpu
from jax.experimental.pallas import tpu_sc as plsc

_NW = 50000
_NS = 10000
_D = 128
_H = 8
_OUT = 4

_f32 = jnp.float32
_i32 = jnp.int32


def _mm2_body(y, wl, bl, wr, br, xl, xr):
    yv = y[...]
    xl[...] = jnp.dot(yv, wl[...], preferred_element_type=_f32) + bl[...]
    xr[...] = jnp.dot(yv, wr[...], preferred_element_type=_f32) + br[...]


def _mm2(y, p):
    n = y.shape[0]
    bn = 1000
    return pl.pallas_call(
        _mm2_body,
        grid=(n // bn,),
        in_specs=[
            pl.BlockSpec((bn, _D), lambda i: (i, 0)),
            pl.BlockSpec((_D, _D), lambda i: (0, 0)),
            pl.BlockSpec((1, _D), lambda i: (0, 0)),
            pl.BlockSpec((_D, _D), lambda i: (0, 0)),
            pl.BlockSpec((1, _D), lambda i: (0, 0)),
        ],
        out_specs=[pl.BlockSpec((bn, _D), lambda i: (i, 0))] * 2,
        out_shape=[jax.ShapeDtypeStruct((n, _D), _f32)] * 2,
    )(y, p['Wl'], p['bl'][None], p['Wr'], p['br'][None])


def _xe_body(ea, we, xe, cs):
    x = jnp.dot(ea[...], we[...], preferred_element_type=_f32)
    xe[...] = x

    @pl.when(pl.program_id(0) == 0)
    def _():
        cs[...] = jnp.zeros_like(cs)

    cs[...] += jnp.sum(x, axis=0, keepdims=True)


def _xe(ea_pad, we):
    ep = ea_pad.shape[0]
    be = 512
    return pl.pallas_call(
        _xe_body,
        grid=(ep // be,),
        in_specs=[
            pl.BlockSpec((be, 17), lambda i: (i, 0)),
            pl.BlockSpec((17, _D), lambda i: (0, 0)),
        ],
        out_specs=[
            pl.BlockSpec((be, _D), lambda i: (i, 0)),
            pl.BlockSpec((1, _D), lambda i: (0, 0)),
        ],
        out_shape=[
            jax.ShapeDtypeStruct((ep, _D), _f32),
            jax.ShapeDtypeStruct((1, _D), _f32),
        ],
    )(ea_pad, we)


def _combine_body(p0, p1, d0, d1, xl, xr, fill, attb, b8, bias, g, b, out):
    xlv = xl[...]
    m = xlv + xr[...] + fill[...]
    ma = jnp.maximum(m, 0.2 * m)
    exs = jnp.exp(jnp.dot(ma, attb[...], preferred_element_type=_f32))
    den = d0[...][:, :8] + d1[...][:, :8] + exs
    acc = p0[...] + p1[...] + xlv * jnp.dot(exs, b8[...], preferred_element_type=_f32)
    o = acc / (jnp.dot(den, b8[...], preferred_element_type=_f32) + 1e-16)
    o = o + bias[...]
    mu = jnp.mean(o, axis=1, keepdims=True)
    var = jnp.mean(jnp.square(o - mu), axis=1, keepdims=True)
    o = (o - mu) * lax.rsqrt(var + 1e-5) * g[...] + b[...]
    out[...] = jnp.maximum(o, 0.0)


def _combine(p0, p1, d0, d1, xl, xr, fill, attb, b8, p, lnp):
    n = xl.shape[0]
    bn = 1000
    return pl.pallas_call(
        _combine_body,
        grid=(n // bn,),
        in_specs=[
            pl.BlockSpec((bn, _D), lambda i: (i, 0)),
            pl.BlockSpec((bn, _D), lambda i: (i, 0)),
            pl.BlockSpec((bn, 16), lambda i: (i, 0)),
            pl.BlockSpec((bn, 16), lambda i: (i, 0)),
            pl.BlockSpec((bn, _D), lambda i: (i, 0)),
            pl.BlockSpec((bn, _D), lambda i: (i, 0)),
            pl.BlockSpec((1, _D), lambda i: (0, 0)),
            pl.BlockSpec((_D, _H), lambda i: (0, 0)),
            pl.BlockSpec((_H, _D), lambda i: (0, 0)),
            pl.BlockSpec((1, _D), lambda i: (0, 0)),
            pl.BlockSpec((1, _D), lambda i: (0, 0)),
            pl.BlockSpec((1, _D), lambda i: (0, 0)),
        ],
        out_specs=pl.BlockSpec((bn, _D), lambda i: (i, 0)),
        out_shape=jax.ShapeDtypeStruct((n, _D), _f32),
    )(p0, p1, d0, d1, xl, xr, fill, attb, b8, p['bias'][None],
      lnp['g'][None], lnp['b'][None])


def _cls_body(y, w1, b1, g, b, w2, b2, out):
    h = jnp.dot(y[...], w1[...], preferred_element_type=_f32) + b1[...]
    mu = jnp.mean(h, axis=1, keepdims=True)
    var = jnp.mean(jnp.square(h - mu), axis=1, keepdims=True)
    h = (h - mu) * lax.rsqrt(var + 1e-5) * g[...] + b[...]
    h = jnp.maximum(h, 0.0)
    out[...] = jnp.dot(h, w2[...], preferred_element_type=_f32) + b2[...]


def _cls(y, params):
    n = y.shape[0]
    bn = 1000
    dq = _D // 4
    return pl.pallas_call(
        _cls_body,
        grid=(n // bn,),
        in_specs=[
            pl.BlockSpec((bn, _D), lambda i: (i, 0)),
            pl.BlockSpec((_D, dq), lambda i: (0, 0)),
            pl.BlockSpec((1, dq), lambda i: (0, 0)),
            pl.BlockSpec((1, dq), lambda i: (0, 0)),
            pl.BlockSpec((1, dq), lambda i: (0, 0)),
            pl.BlockSpec((dq, _OUT), lambda i: (0, 0)),
            pl.BlockSpec((1, _OUT), lambda i: (0, 0)),
        ],
        out_specs=pl.BlockSpec((bn, _OUT), lambda i: (i, 0)),
        out_shape=jax.ShapeDtypeStruct((n, _OUT), _f32),
    )(y, params['cls_W1'], params['cls_b1'][None], params['cls_ln']['g'][None],
      params['cls_ln']['b'][None], params['cls_W2'], params['cls_b2'][None])


def _edge_call(src, dst, xl, xr, xe, att2, ep, n, nr, r_cnt):
    has_xe = xe is not None
    ew = ep // 32
    ng = ew // 16
    maxc = ((ew + 63) // 64) * 64 + 128
    nrp = nr + 512
    rpt = nrp // 16
    zb = 64
    nd = nr // 8
    npad = nr * r_cnt

    mesh = plsc.VectorSubcoreMesh(core_axis_name="c", subcore_axis_name="s")

    scratch = [
        pltpu.VMEM((ew,), _i32),
        pltpu.VMEM((ew,), _i32),
        pltpu.VMEM((maxc,), _i32),
        pltpu.VMEM((64,), _i32),
        pltpu.VMEM((64,), _i32),
        pltpu.VMEM((1, 64), _i32),
        pltpu.VMEM((1, 64), _i32),
        pltpu.VMEM((64, _D), _f32),
        pltpu.VMEM((64, _D), _f32),
        pltpu.VMEM((64, _D), _f32),
        pltpu.VMEM((64, _D), _f32),
        pltpu.VMEM((_H, 16), _f32),
        pltpu.VMEM((zb, _D), _f32),
        pltpu.VMEM_SHARED((nrp, _D), _f32),
        pltpu.VMEM_SHARED((nr // 8 + 128, _D), _f32),
        pltpu.SemaphoreType.DMA,
        pltpu.SemaphoreType.DMA,
        pltpu.SemaphoreType.DMA,
    ]
    if has_xe:
        scratch = scratch[:7] + [pltpu.VMEM((64,), _i32),
                                 pltpu.VMEM((64, _D), _f32)] + scratch[7:]

    def body(*refs):
        if has_xe:
            (src_h, dst_h, xl_h, xr_h, xe_h, att_h, outp_h, denp_h,
             srcv, dstv, ceid, cidxs, cidx, cdst2, cdst3, cidxe, xeb,
             xlb, xrb, sbuf, sbuf2, attv, zbuf, acc, acc2,
             sem1, sem2, sem3) = refs
        else:
            (src_h, dst_h, xl_h, xr_h, att_h, outp_h, denp_h,
             srcv, dstv, ceid, cidxs, cidx, cdst2, cdst3,
             xlb, xrb, sbuf, sbuf2, attv, zbuf, acc, acc2,
             sem1, sem2, sem3) = refs

        cid = lax.axis_index("c")
        sid = lax.axis_index("s")
        wid = sid * 2 + cid
        base = wid * ew
        iota16 = lax.iota(_i32, 16)
        zeros16 = jnp.zeros((16,), _f32)
        shuf_idx = [jnp.maximum(iota16 - k, 0)[:, None] for k in (1, 2, 4, 8)]
        shuf_keep = [iota16 >= k for k in (1, 2, 4, 8)]
        trash = maxc - 16 + iota16
        gd = lax.GatherDimensionNumbers(
            offset_dims=(), collapsed_slice_dims=(0,), start_index_map=(0,))

        def _lane_shift(v, idx):
            return lax.gather(v, idx, gd, (1,),
                              mode=lax.GatherScatterMode.PROMISE_IN_BOUNDS)

        pltpu.sync_copy(src_h.at[pl.ds(base, ew)], srcv)
        pltpu.sync_copy(dst_h.at[pl.ds(base, ew)], dstv)
        pltpu.sync_copy(att_h, attv)

        def _zrow(i, c):
            for j in range(8):
                zbuf[i, pl.ds(16 * j, 16)] = zeros16
            return c
        lax.fori_loop(0, zb, _zrow, 0)

        def _spad(e, c):
            for j in range(8):
                sbuf2[e, pl.ds(16 * j, 16)] = zeros16
            return c
        lax.fori_loop(0, 64, _spad, 0)
        attr = [attv[h] for h in range(8)]

        def _range_body(r, carry):
            lo = r * nr

            def _zc(i, c):
                pltpu.sync_copy(zbuf, acc.at[pl.ds(sid * rpt + i * zb, zb)])
                return c
            lax.fori_loop(0, rpt // zb, _zc, 0)
            if rpt % zb:
                pltpu.sync_copy(
                    zbuf.at[pl.ds(0, rpt % zb)],
                    acc.at[pl.ds(sid * rpt + (rpt // zb) * zb, rpt % zb)])
            d16 = (nd + 128) // 16
            pltpu.sync_copy(zbuf.at[pl.ds(0, d16)],
                            acc2.at[pl.ds(sid * d16, d16)])
            plsc.subcore_barrier()

            def _pfx(d):
                dl = d - lo
                msk = (dl >= 0) & (dl < nr)
                pf = jnp.where(msk, 1, 0).astype(_i32)
                for kk in range(4):
                    sh = _lane_shift(pf, shuf_idx[kk])
                    pf = pf + jnp.where(shuf_keep[kk], sh, 0)
                return msk, pf

            def _cg(g2, cnt):
                ma, pa = _pfx(dstv[pl.ds(g2 * 32, 16)])
                mb, pb = _pfx(dstv[pl.ds(g2 * 32 + 16, 16)])
                ca = pa[15]
                plsc.store_scatter(
                    ceid, [jnp.where(ma, cnt + pa - 1, trash)],
                    g2 * 32 + iota16)
                plsc.store_scatter(
                    ceid, [jnp.where(mb, cnt + ca + pb - 1, trash)],
                    g2 * 32 + 16 + iota16)
                return cnt + ca + pb[15]
            cnt = lax.fori_loop(0, ng // 2, _cg, jnp.array(0, _i32))
            if ng % 2:
                g = ng - 1
                mt, pt = _pfx(dstv[pl.ds(g * 16, 16)])
                plsc.store_scatter(
                    ceid, [jnp.where(mt, cnt + pt - 1, trash)],
                    g * 16 + iota16)
                cnt = cnt + pt[15]
            nchunks = (cnt + 63) // 64

            def _ck(k, c):
                for j in range(4):
                    posv = k * 64 + 16 * j + iota16
                    eidj = jnp.clip(ceid[pl.ds(k * 64 + 16 * j, 16)],
                                    0, ew - 1)
                    srcj = plsc.load_gather(srcv, [eidj])
                    dstj = plsc.load_gather(dstv, [eidj])
                    dlj = jnp.where(posv < cnt, dstj - lo, nr)
                    cidxs[pl.ds(16 * j, 16)] = srcj
                    cdst2[0, pl.ds(16 * j, 16)] = dlj
                    cdst3[0, pl.ds(16 * j, 16)] = dlj >> 3
                    cidx[pl.ds(16 * j, 16)] = jnp.minimum(dlj + lo, n - 1)
                    if has_xe:
                        cidxe[pl.ds(16 * j, 16)] = eidj + base
                cp1 = pltpu.async_copy(xl_h.at[cidxs], xlb, sem1)
                cp2 = pltpu.async_copy(xr_h.at[cidx], xrb, sem2)
                if has_xe:
                    cp3 = pltpu.async_copy(xe_h.at[cidxe], xeb, sem3)
                cp1.wait()
                cp2.wait()
                if has_xe:
                    cp3.wait()

                def _ed(e2, cc):
                    es = [e2 * 2, e2 * 2 + 1]
                    cbs = []
                    for e in es:
                        dlsp = plsc.load_gather(
                            cdst2, [jnp.zeros((16,), _i32),
                                    jnp.full((16,), 0, _i32) + e])
                        cbs.append(((dlsp & 7) << 4)[0])
                    xls = [[xlb[e, pl.ds(16 * j, 16)] for j in range(8)]
                           for e in es]
                    ts = []
                    for i, e in enumerate(es):
                        for h in range(8):
                            mv = xls[i][h] + xrb[e, pl.ds(16 * h, 16)]
                            if has_xe:
                                mv = mv + xeb[e, pl.ds(16 * h, 16)]
                            ts.append(jnp.maximum(mv, 0.2 * mv) * attr[h])
                    als = [jnp.sum(t) for t in ts]
                    exvs = [jnp.exp(jnp.full((16,), a, _f32)) for a in als]
                    for i, e in enumerate(es):
                        packed = zeros16
                        for h in range(8):
                            exv = exvs[i * 8 + h]
                            sbuf[e, pl.ds(16 * h, 16)] = xls[i][h] * exv
                            packed = jnp.where(iota16 == h, exv, packed)
                        sbuf2[e, pl.ds(cbs[i], 16)] = packed
                    return cc
                lax.fori_loop(0, 32, _ed, 0)

                pltpu.sync_copy(sbuf, acc.at[cdst2.at[0]], add=True)
                pltpu.sync_copy(sbuf2, acc2.at[cdst3.at[0]], add=True)

                def _zs(e, cc):
                    dlsp = plsc.load_gather(
                        cdst2, [jnp.zeros((16,), _i32),
                                jnp.full((16,), 0, _i32) + e])
                    cb = ((dlsp & 7) << 4)[0]
                    sbuf2[e, pl.ds(cb, 16)] = zeros16
                    return cc
                lax.fori_loop(0, 64, _zs, 0)
                return c
            lax.fori_loop(0, nchunks, _ck, 0)

            plsc.subcore_barrier()

            row0 = sid * (nr // 16)
            pltpu.sync_copy(acc.at[pl.ds(row0, nr // 16)],
                            outp_h.at[cid, pl.ds(lo + row0, nr // 16)])
            row0d = sid * (nd // 16)
            pltpu.sync_copy(acc2.at[pl.ds(row0d, nd // 16)],
                            denp_h.at[cid, pl.ds(r * nd + row0d, nd // 16)])
            plsc.subcore_barrier()
            return carry
        lax.fori_loop(0, r_cnt, _range_body, 0)

    out_type = [
        jax.ShapeDtypeStruct((2, npad, 128), _f32),
        jax.ShapeDtypeStruct((2, npad // 8, 128), _f32),
    ]
    args = [src, dst, xl, xr]
    if has_xe:
        args.append(xe)
    args.append(att2)
    return pl.kernel(
        body, out_type=out_type, mesh=mesh, scratch_types=scratch,
        compiler_params=pltpu.CompilerParams(needs_layout_passes=False),
    )(*args)


def _pad_edges(ei, n, ep):
    e = ei.shape[1]
    src = jnp.concatenate([ei[0], jnp.zeros((ep - e,), _i32)])
    dst = jnp.concatenate([ei[1], jnp.full((ep - e,), n, _i32)])
    return src, dst


def _gat_layer(p, y, src, dst, ep, n, nr, r_cnt, b8, xe=None, fill=None):
    xl, xr = _mm2(y, p)
    att2 = p['att'].reshape(_H, 16)
    attb = jnp.zeros((_D, _H), _f32).at[
        jnp.arange(_D), jnp.arange(_D) // 16].set(p['att'].reshape(_D))
    outp, denp = _edge_call(src, dst, xl, xr, xe, att2, ep, n, nr, r_cnt)
    denr = denp.reshape(2, nr * r_cnt, 16)
    if fill is None:
        fill = jnp.zeros((1, _D), _f32)
    return _combine(outp[0, :n], outp[1, :n], denr[0, :n], denr[1, :n],
                    xl, xr, fill, attb, b8, p, p['_ln'])


def kernel(x_words, x_sent, w2w_index, w2s_index, s2s_index, s2s_type,
           s2s_sim, params):
    b8 = (jnp.arange(_D)[None, :] // 16 ==
          jnp.arange(_H)[:, None]).astype(_f32)

    ew_p = ((w2w_index.shape[1] + 511) // 512) * 512
    ews_p = ((w2s_index.shape[1] + 511) // 512) * 512
    es_p = ((s2s_index.shape[1] + 511) // 512) * 512

    w2w_s, w2w_d = _pad_edges(w2w_index, _NW, ew_p)
    w2s_s, w2s_d = _pad_edges(w2s_index, _NW + _NS, ews_p)
    s2s_s, s2s_d = _pad_edges(s2s_index, _NS, es_p)
    sim_s, sim_d = _pad_edges(s2s_sim, _NS, es_p)

    es = s2s_index.shape[1]
    ea_pad = jnp.concatenate(
        [s2s_type, jnp.zeros((es_p - es, 17), _f32)], axis=0)

    def lp(name, ln):
        q = dict(params[name])
        q['_ln'] = params[ln]
        return q

    y = _gat_layer(lp('w2w_1', 'ln1'), x_words, w2w_s, w2w_d,
                   ew_p, _NW, 4096, 13, b8)
    y = _gat_layer(lp('w2w_1', 'ln2'), y, w2w_s, w2w_d,
                   ew_p, _NW, 4096, 13, b8)
    yc = jnp.concatenate([y, x_sent], axis=0)
    yc = _gat_layer(lp('word_to_sent', 'ln3'), yc, w2s_s, w2s_d,
                    ews_p, _NW + _NS, 4096, 15, b8)
    ys = yc[_NW:]

    xe1, cs1 = _xe(ea_pad, params['s2s_1']['We'])
    ys = _gat_layer(lp('s2s_1', 'ln4'), ys, s2s_s, s2s_d,
                    es_p, _NS, 4096, 3, b8, xe=xe1, fill=cs1 / es)
    xe2, cs2 = _xe(ea_pad, params['s2s_2']['We'])
    ys = _gat_layer(lp('s2s_2', 'ln5'), ys, s2s_s, s2s_d,
                    es_p, _NS, 4096, 3, b8, xe=xe2, fill=cs2 / es)

    ys = _gat_layer(lp('red_1', 'ln6'), ys, sim_s, sim_d,
                    es_p, _NS, 4096, 3, b8)
    ys = _gat_layer(lp('red_2', 'ln7'), ys, sim_s, sim_d,
                    es_p, _NS, 4096, 3, b8)

    return _cls(ys, params)
